# baseline (speedup 1.0000x reference)
import functools

import jax
import jax.numpy as jnp
from jax import lax
from jax.experimental import pallas as pl
from jax.experimental.pallas import tpu as pltpu
from jax.experimental.pallas import tpu_sc as plsc

EMB = 256
HEADS = 8
DH = EMB // HEADS
NNB = 16
N = 10000

NC, NS = 2, 16
NW = NC * NS
EDGES = N * NNB
CHUNK = 128
NCH0 = 8
NCH1 = 72
EPAD = NS * (NCH0 + NCH1) * CHUNK

BN = 200
GRID = N // BN


def _sc_gather_body(x_hbm, idx_hbm, out_hbm, idx_v, rows0, rows1, gsem0, gsem1):
    cid = lax.axis_index("c")
    sid = lax.axis_index("s")

    def pipeline(base, nchunk):
        pltpu.sync_copy(idx_hbm.at[pl.ds(base, nchunk * CHUNK)],
                        idx_v.at[pl.ds(0, nchunk * CHUNK)])

        def gidx(g):
            return idx_v.at[pl.ds(g * CHUNK, CHUNK)]

        pltpu.async_copy(x_hbm.at[gidx(0)], rows0, gsem0)
        pltpu.async_copy(x_hbm.at[gidx(1)], rows1, gsem1)

        def iter2(i, carry):
            t = 2 * i
            for b, rows, gsem in ((0, rows0, gsem0), (1, rows1, gsem1)):
                g = t + b
                pltpu.make_async_copy(x_hbm.at[gidx(g)], rows, gsem).wait()
                pltpu.sync_copy(rows, out_hbm.at[pl.ds(base + g * CHUNK, CHUNK)])
                pltpu.async_copy(x_hbm.at[gidx(g + 2)], rows, gsem)
            return carry

        lax.fori_loop(0, (nchunk - 2) // 2, iter2, 0)

        for b, rows, gsem in ((0, rows0, gsem0), (1, rows1, gsem1)):
            g = nchunk - 2 + b
            pltpu.make_async_copy(x_hbm.at[gidx(g)], rows, gsem).wait()
            pltpu.sync_copy(rows, out_hbm.at[pl.ds(base + g * CHUNK, CHUNK)])

    @pl.when(cid == 0)
    def _():
        pipeline(sid * (NCH0 * CHUNK), NCH0)

    @pl.when(cid == 1)
    def _():
        pipeline(NS * NCH0 * CHUNK + sid * (NCH1 * CHUNK), NCH1)


@functools.cache
def _sc_gather():
    return pl.kernel(
        _sc_gather_body,
        out_type=jax.ShapeDtypeStruct((EPAD, EMB), jnp.float32),
        mesh=plsc.VectorSubcoreMesh(
            core_axis_name="c", subcore_axis_name="s",
            num_cores=NC, num_subcores=NS,
        ),
        scratch_types=[
            pltpu.VMEM((max(NCH0, NCH1) * CHUNK,), jnp.int32),
            pltpu.VMEM((CHUNK, EMB), jnp.float32),
            pltpu.VMEM((CHUNK, EMB), jnp.float32),
            pltpu.SemaphoreType.DMA,
            pltpu.SemaphoreType.DMA,
        ],
    )


def _attn_body(x_ref, xg_ref, wq_ref, bq_ref, wk_ref, bk_ref, wv_ref, bv_ref,
               out_ref, att_ref):
    f32 = jnp.float32
    x = x_ref[...]
    xg = xg_ref[...]
    q = jnp.dot(x, wq_ref[...], preferred_element_type=f32) + bq_ref[...]
    kg = jnp.dot(xg, wk_ref[...], preferred_element_type=f32) + bk_ref[...]
    vg = jnp.dot(xg, wv_ref[...], preferred_element_type=f32) + bv_ref[...]

    col = lax.broadcasted_iota(jnp.int32, (EMB, HEADS), 0) // DH
    head = lax.broadcasted_iota(jnp.int32, (EMB, HEADS), 1)
    hm = (col == head).astype(f32)

    qrep = jnp.broadcast_to(q[:, None, :], (BN, NNB, EMB)).reshape(BN * NNB, EMB)
    e8 = jnp.dot(qrep * kg, hm, preferred_element_type=f32)
    e3 = e8.reshape(BN, NNB, HEADS)

    m = jnp.max(e3, axis=1, keepdims=True)
    p = jnp.exp(e3 - m)
    s = jnp.sum(p, axis=1, keepdims=True)
    att3 = p / (s * 16.0)

    att_ref[...] = jnp.sum(att3, axis=(1, 2)).reshape(BN, 1)

    attexp = jnp.dot(att3.reshape(BN * NNB, HEADS), hm.T,
                     preferred_element_type=f32)
    out3 = (attexp * vg).reshape(BN, NNB, EMB)
    out_ref[...] = jnp.sum(out3, axis=1)


def _attention(x2, xg, wqt, bq2, wkt, bk2, wvt, bv2):
    return pl.pallas_call(
        _attn_body,
        grid=(GRID,),
        in_specs=[
            pl.BlockSpec((BN, EMB), lambda i: (i, 0)),
            pl.BlockSpec((BN * NNB, EMB), lambda i: (i, 0)),
            pl.BlockSpec((EMB, EMB), lambda i: (0, 0)),
            pl.BlockSpec((1, EMB), lambda i: (0, 0)),
            pl.BlockSpec((EMB, EMB), lambda i: (0, 0)),
            pl.BlockSpec((1, EMB), lambda i: (0, 0)),
            pl.BlockSpec((EMB, EMB), lambda i: (0, 0)),
            pl.BlockSpec((1, EMB), lambda i: (0, 0)),
        ],
        out_specs=[
            pl.BlockSpec((BN, EMB), lambda i: (i, 0)),
            pl.BlockSpec((BN, 1), lambda i: (i, 0)),
        ],
        out_shape=[
            jax.ShapeDtypeStruct((N, EMB), jnp.float32),
            jax.ShapeDtypeStruct((N, 1), jnp.float32),
        ],
    )(x2, xg, wqt, bq2, wkt, bk2, wvt, bv2)


def kernel(x, A, Wq, bq, Wk, bk, Wv, bv):
    b, n, e = x.shape
    x2 = x.reshape(n, e)
    idx = A.reshape(-1).astype(jnp.int32)
    idx_pad = jnp.concatenate(
        [idx, jnp.zeros((EPAD - EDGES,), dtype=jnp.int32)])
    xg = _sc_gather()(x2, idx_pad)
    l, att = _attention(
        x2, xg,
        Wq.T, bq.reshape(1, e),
        Wk.T, bk.reshape(1, e),
        Wv.T, bv.reshape(1, e),
    )
    return l.reshape(b, n, e), att


# speedup vs baseline: 1.0956x; 1.0956x over previous
import functools

import jax
import jax.numpy as jnp
from jax import lax
from jax.experimental import pallas as pl
from jax.experimental.pallas import tpu as pltpu
from jax.experimental.pallas import tpu_sc as plsc

EMB = 256
HEADS = 8
DH = EMB // HEADS
NNB = 16
N = 10000

NC, NS = 2, 16
NW = NC * NS
EDGES = N * NNB
CHUNK = 128
NCH0 = 72
NCH1 = 8
EPAD = NS * (NCH0 + NCH1) * CHUNK

BN = 200
GRID = N // BN


def _sc_gather_body(x_hbm, idx_hbm, out_hbm, idx_v, rows0, rows1, gsem0, gsem1):
    cid = lax.axis_index("c")
    sid = lax.axis_index("s")

    def pipeline(base, nchunk):
        pltpu.sync_copy(idx_hbm.at[pl.ds(base, nchunk * CHUNK)],
                        idx_v.at[pl.ds(0, nchunk * CHUNK)])

        def gidx(g):
            return idx_v.at[pl.ds(g * CHUNK, CHUNK)]

        pltpu.async_copy(x_hbm.at[gidx(0)], rows0, gsem0)
        pltpu.async_copy(x_hbm.at[gidx(1)], rows1, gsem1)

        def iter2(i, carry):
            t = 2 * i
            for b, rows, gsem in ((0, rows0, gsem0), (1, rows1, gsem1)):
                g = t + b
                pltpu.make_async_copy(x_hbm.at[gidx(g)], rows, gsem).wait()
                pltpu.sync_copy(rows, out_hbm.at[pl.ds(base + g * CHUNK, CHUNK)])
                pltpu.async_copy(x_hbm.at[gidx(g + 2)], rows, gsem)
            return carry

        lax.fori_loop(0, (nchunk - 2) // 2, iter2, 0)

        for b, rows, gsem in ((0, rows0, gsem0), (1, rows1, gsem1)):
            g = nchunk - 2 + b
            pltpu.make_async_copy(x_hbm.at[gidx(g)], rows, gsem).wait()
            pltpu.sync_copy(rows, out_hbm.at[pl.ds(base + g * CHUNK, CHUNK)])

    @pl.when(cid == 0)
    def _():
        pipeline(sid * (NCH0 * CHUNK), NCH0)

    @pl.when(cid == 1)
    def _():
        pipeline(NS * NCH0 * CHUNK + sid * (NCH1 * CHUNK), NCH1)


@functools.cache
def _sc_gather():
    return pl.kernel(
        _sc_gather_body,
        out_type=jax.ShapeDtypeStruct((EPAD, EMB), jnp.float32),
        mesh=plsc.VectorSubcoreMesh(
            core_axis_name="c", subcore_axis_name="s",
            num_cores=NC, num_subcores=NS,
        ),
        scratch_types=[
            pltpu.VMEM((max(NCH0, NCH1) * CHUNK,), jnp.int32),
            pltpu.VMEM((CHUNK, EMB), jnp.float32),
            pltpu.VMEM((CHUNK, EMB), jnp.float32),
            pltpu.SemaphoreType.DMA,
            pltpu.SemaphoreType.DMA,
        ],
    )


def _attn_body(x_ref, xg_ref, wq_ref, bq_ref, wk_ref, bk_ref, wv_ref, bv_ref,
               out_ref, att_ref):
    f32 = jnp.float32
    x = x_ref[...]
    xg = xg_ref[...]
    q = jnp.dot(x, wq_ref[...], preferred_element_type=f32) + bq_ref[...]
    kg = jnp.dot(xg, wk_ref[...], preferred_element_type=f32) + bk_ref[...]
    vg = jnp.dot(xg, wv_ref[...], preferred_element_type=f32) + bv_ref[...]

    col = lax.broadcasted_iota(jnp.int32, (EMB, HEADS), 0) // DH
    head = lax.broadcasted_iota(jnp.int32, (EMB, HEADS), 1)
    hm = (col == head).astype(f32)

    qrep = jnp.broadcast_to(q[:, None, :], (BN, NNB, EMB)).reshape(BN * NNB, EMB)
    e8 = jnp.dot(qrep * kg, hm, preferred_element_type=f32)
    e3 = e8.reshape(BN, NNB, HEADS)

    m = jnp.max(e3, axis=1, keepdims=True)
    p = jnp.exp(e3 - m)
    s = jnp.sum(p, axis=1, keepdims=True)
    att3 = p / (s * 16.0)

    att_ref[...] = jnp.sum(att3, axis=(1, 2)).reshape(BN, 1)

    attexp = jnp.dot(att3.reshape(BN * NNB, HEADS), hm.T,
                     preferred_element_type=f32)
    out3 = (attexp * vg).reshape(BN, NNB, EMB)
    out_ref[...] = jnp.sum(out3, axis=1)


def _attention(x2, xg, wqt, bq2, wkt, bk2, wvt, bv2):
    return pl.pallas_call(
        _attn_body,
        grid=(GRID,),
        in_specs=[
            pl.BlockSpec((BN, EMB), lambda i: (i, 0)),
            pl.BlockSpec((BN * NNB, EMB), lambda i: (i, 0)),
            pl.BlockSpec((EMB, EMB), lambda i: (0, 0)),
            pl.BlockSpec((1, EMB), lambda i: (0, 0)),
            pl.BlockSpec((EMB, EMB), lambda i: (0, 0)),
            pl.BlockSpec((1, EMB), lambda i: (0, 0)),
            pl.BlockSpec((EMB, EMB), lambda i: (0, 0)),
            pl.BlockSpec((1, EMB), lambda i: (0, 0)),
        ],
        out_specs=[
            pl.BlockSpec((BN, EMB), lambda i: (i, 0)),
            pl.BlockSpec((BN, 1), lambda i: (i, 0)),
        ],
        out_shape=[
            jax.ShapeDtypeStruct((N, EMB), jnp.float32),
            jax.ShapeDtypeStruct((N, 1), jnp.float32),
        ],
    )(x2, xg, wqt, bq2, wkt, bk2, wvt, bv2)


def kernel(x, A, Wq, bq, Wk, bk, Wv, bv):
    b, n, e = x.shape
    x2 = x.reshape(n, e)
    idx = A.reshape(-1).astype(jnp.int32)
    idx_pad = jnp.concatenate(
        [idx, jnp.zeros((EPAD - EDGES,), dtype=jnp.int32)])
    xg = _sc_gather()(x2, idx_pad)
    l, att = _attention(
        x2, xg,
        Wq.T, bq.reshape(1, e),
        Wk.T, bk.reshape(1, e),
        Wv.T, bv.reshape(1, e),
    )
    return l.reshape(b, n, e), att
